# Initial kernel scaffold; baseline (speedup 1.0000x reference)
import jax
import jax.numpy as jnp
from jax.experimental import pallas as pl


def kernel(node_feats, edge_feats, edge_index, W1, b1, W2, b2):
    raise NotImplementedError("write your pallas kernel here")


# speedup vs baseline: 3.9349x; 3.9349x over previous
import functools

import jax
import jax.numpy as jnp
from jax import lax
from jax.experimental import pallas as pl
from jax.experimental.pallas import tpu as pltpu
from jax.experimental.pallas import tpu_sc as plsc

N = 10000
E = 320000
D = 128
H = 128
O = 128

NC = 2
NS = 16
NW = NC * NS
EPW = E // NW
K = 80
STEPS = EPW // K
RPT = 624
TAIL = N - RPT * NS


def _sc_body(node_hbm, ef_hbm, src_hbm, dst_hbm, zeros_hbm, part_hbm,
             src_v, dst_v, rows_v, ef_v, acc, sem):
    c = lax.axis_index("c")
    s = lax.axis_index("s")
    wid = c * NS + s
    base = wid * EPW

    pltpu.sync_copy(zeros_hbm.at[pl.ds(s * RPT, RPT)],
                    acc.at[pl.ds(s * RPT, RPT)])
    @pl.when(s == NS - 1)
    def _():
        pltpu.sync_copy(zeros_hbm.at[pl.ds(RPT * NS, TAIL)],
                        acc.at[pl.ds(RPT * NS, TAIL)])
    plsc.subcore_barrier()

    def step(i, carry):
        off = base + i * K
        pltpu.sync_copy(src_hbm.at[pl.ds(off, K)], src_v)
        pltpu.sync_copy(dst_hbm.at[pl.ds(off, K)], dst_v)
        gat = pltpu.async_copy(node_hbm.at[src_v], rows_v, sem)
        pltpu.sync_copy(ef_hbm.at[pl.ds(off, K)], ef_v)
        gat.wait()

        def addrow(r, c2):
            for j in range(D // 16):
                sl = pl.ds(j * 16, 16)
                rows_v[r, sl] = rows_v[r, sl] + ef_v[r, sl]
            return c2
        lax.fori_loop(0, K, addrow, 0)

        pltpu.sync_copy(rows_v, acc.at[dst_v], add=True)
        return carry

    lax.fori_loop(0, STEPS, step, 0)
    plsc.subcore_barrier()

    pltpu.sync_copy(acc.at[pl.ds(s * RPT, RPT)],
                    part_hbm.at[c, pl.ds(s * RPT, RPT)])
    @pl.when(s == NS - 1)
    def _():
        pltpu.sync_copy(acc.at[pl.ds(RPT * NS, TAIL)],
                        part_hbm.at[c, pl.ds(RPT * NS, TAIL)])


_sc_segment_sum = pl.kernel(
    _sc_body,
    out_type=jax.ShapeDtypeStruct((NC, N, D), jnp.float32),
    mesh=plsc.VectorSubcoreMesh(core_axis_name="c", subcore_axis_name="s"),
    scratch_types=[
        pltpu.VMEM((K,), jnp.int32),
        pltpu.VMEM((K,), jnp.int32),
        pltpu.VMEM((K, D), jnp.float32),
        pltpu.VMEM((K, D), jnp.float32),
        pltpu.VMEM_SHARED((N, D), jnp.float32),
        pltpu.SemaphoreType.DMA,
    ],
)


BLK = 1000


def _mlp_body(p0_ref, p1_ref, w1_ref, b1_ref, w2_ref, b2_ref, o_ref):
    x = p0_ref[...] + p1_ref[...]
    h = jnp.dot(x, w1_ref[...], preferred_element_type=jnp.float32)
    h = jnp.maximum(h + b1_ref[...], 0.0)
    o = jnp.dot(h, w2_ref[...], preferred_element_type=jnp.float32)
    o_ref[...] = o + b2_ref[...]


_mlp = pl.pallas_call(
    _mlp_body,
    grid=(N // BLK,),
    in_specs=[
        pl.BlockSpec((BLK, D), lambda i: (i, 0)),
        pl.BlockSpec((BLK, D), lambda i: (i, 0)),
        pl.BlockSpec((D, H), lambda i: (0, 0)),
        pl.BlockSpec((1, H), lambda i: (0, 0)),
        pl.BlockSpec((H, O), lambda i: (0, 0)),
        pl.BlockSpec((1, O), lambda i: (0, 0)),
    ],
    out_specs=pl.BlockSpec((BLK, O), lambda i: (i, 0)),
    out_shape=jax.ShapeDtypeStruct((N, O), jnp.float32),
)


@jax.jit
def kernel(node_feats, edge_feats, edge_index, W1, b1, W2, b2):
    src = edge_index[0]
    dst = edge_index[1]
    zeros = jnp.zeros((N, D), jnp.float32)
    parts = _sc_segment_sum(node_feats, edge_feats, src, dst, zeros)
    return _mlp(parts[0], parts[1], W1, b1.reshape(1, H), W2, b2.reshape(1, O))
